# Initial kernel scaffold; baseline (speedup 1.0000x reference)
import jax
import jax.numpy as jnp
from jax.experimental import pallas as pl


def kernel(x, routing_scores, expert_w, expert_b):
    raise NotImplementedError("write your pallas kernel here")


# speedup vs baseline: 1.6179x; 1.6179x over previous
import functools

import jax
import jax.numpy as jnp
from jax.experimental import pallas as pl
from jax.experimental.pallas import tpu as pltpu

NEG_INF = float("-inf")


def _fused_moe_kernel(rs_ref, x_ref, w_ref, b_ref, out_ref, counts_ref, ms_ref):
    e = pl.program_id(0)

    @pl.when(e == 0)
    def _init():
        rs = rs_ref[...]
        E = rs.shape[-1]
        lane = jax.lax.broadcasted_iota(jnp.int32, rs.shape, 1)
        v0 = jnp.max(rs, axis=-1, keepdims=True)
        i0 = jnp.argmax(rs, axis=-1, keepdims=True)
        masked = jnp.where(lane == i0, NEG_INF, rs)
        v1 = jnp.max(masked, axis=-1, keepdims=True)
        i1 = jnp.argmax(masked, axis=-1, keepdims=True)
        mask = (lane == i0) | (lane == i1)
        scale = 1.0 / (v0 + v1 + 1e-08)
        ms = scale * rs * mask.astype(jnp.float32)
        ms_ref[...] = ms
        counts_ref[...] = jnp.sum(mask.astype(jnp.int32), axis=0, keepdims=True)
        out_ref[...] = jnp.dot(ms, b_ref[...], preferred_element_type=jnp.float32)

    w = w_ref[0]
    x = x_ref[...]
    y = jnp.dot(x, w, preferred_element_type=jnp.float32)
    ms = ms_ref[...]
    lane = jax.lax.broadcasted_iota(jnp.int32, ms.shape, 1)
    col = jnp.sum(jnp.where(lane == e, ms, 0.0), axis=1, keepdims=True)
    out_ref[...] += col * y


@jax.jit
def kernel(x, routing_scores, expert_w, expert_b):
    B, S, D = x.shape
    E = routing_scores.shape[-1]
    x2 = x.reshape(S, D)
    rs = routing_scores.reshape(S, E)

    out, counts = pl.pallas_call(
        _fused_moe_kernel,
        grid=(E,),
        in_specs=[
            pl.BlockSpec((S, E), lambda e: (0, 0)),
            pl.BlockSpec((S, D), lambda e: (0, 0)),
            pl.BlockSpec((1, D, D), lambda e: (e, 0, 0)),
            pl.BlockSpec((E, D), lambda e: (0, 0)),
        ],
        out_specs=[
            pl.BlockSpec((S, D), lambda e: (0, 0)),
            pl.BlockSpec((1, E), lambda e: (0, 0)),
        ],
        out_shape=[
            jax.ShapeDtypeStruct((S, D), jnp.float32),
            jax.ShapeDtypeStruct((1, E), jnp.int32),
        ],
        scratch_shapes=[pltpu.VMEM((S, E), jnp.float32)],
        compiler_params=pltpu.CompilerParams(
            dimension_semantics=("arbitrary",),
        ),
    )(rs, x2, expert_w, expert_b)

    return out.reshape(B, S, D), counts.reshape(E)
